# baseline (speedup 1.0000x reference)
import functools

import jax
import jax.numpy as jnp
from jax.experimental import pallas as pl
from jax.experimental.pallas import tpu as pltpu

B, L, N, D, H, HD = 16, 512, 4096, 256, 4, 64
NC, NR = 18, 6
W = L + 8


def _body(cu_ref, flat_ref, xyz_ref, wout_ref, bout_ref, wpos_ref, bpos_ref,
          wqkv_ref, wo_ref, wrc_ref, brc_ref, coords_ref, classes_ref):
    b = pl.program_id(0)
    s = cu_ref[b]
    e = cu_ref[b + 1]
    ln = e - s

    bpos = bpos_ref[0, :]

    def embed(f, x):
        out = jnp.maximum(
            jax.lax.dot(f, wout_ref[:, :],
                        preferred_element_type=jnp.float32) + bout_ref[0, :],
            0.0)
        pe = jax.lax.dot(x, wpos_ref[:, :], preferred_element_type=jnp.float32)
        return out + pe + bpos

    base = pl.multiple_of(jnp.minimum((s // 8) * 8, N - W), 8)
    d = s - base
    h_win = embed(flat_ref[pl.ds(base, W), :],
                  xyz_ref[pl.ds(base, W), :])
    h_roll = pltpu.roll(h_win, W - d, axis=0)[:L, :]

    row = jax.lax.broadcasted_iota(jnp.int32, (L, 1), 0)
    lcap = jnp.minimum(ln, L)
    valid = row < lcap

    last = jnp.maximum(e - 1, 0)
    l_al = pl.multiple_of((last // 8) * 8, 8)
    sel = jax.lax.broadcasted_iota(jnp.int32, (8, 1), 0) == (last - l_al)
    h8 = embed(flat_ref[pl.ds(l_al, 8), :], xyz_ref[pl.ds(l_al, 8), :])
    h_last = jnp.sum(jnp.where(sel, h8, 0.0), axis=0, keepdims=True)
    repl = jnp.logical_and(ln > L, row == (L - 1))
    h = jnp.where(repl, h_last, jnp.where(valid, h_roll, bpos))

    q = jax.lax.dot(h, wqkv_ref[:, :D], preferred_element_type=jnp.float32)
    k = jax.lax.dot(h, wqkv_ref[:, D:2 * D],
                    preferred_element_type=jnp.float32)
    v = jax.lax.dot(h, wqkv_ref[:, 2 * D:],
                    preferred_element_type=jnp.float32)

    ones_col = jnp.ones((L, 1), jnp.float32)
    kbias = jnp.where(valid, 0.0, -40.0)
    heads = []
    for hh in range(H):
        sl = slice(hh * HD, (hh + 1) * HD)
        qh = jnp.concatenate([q[:, sl], ones_col], axis=1)
        kh = jnp.concatenate([k[:, sl], kbias], axis=1)
        vh = jnp.concatenate([v[:, sl], ones_col], axis=1)
        lg = jax.lax.dot_general(qh, kh, (((1,), (1,)), ((), ())),
                                 preferred_element_type=jnp.float32)
        p = jnp.exp(lg)
        pv = jax.lax.dot(p, vh, preferred_element_type=jnp.float32)
        heads.append(pv[:, :HD] / pv[:, HD:HD + 1])
    ao = jnp.concatenate(heads, axis=-1)
    box = h + jax.lax.dot(ao, wo_ref[:, :], preferred_element_type=jnp.float32)

    rc = jax.lax.dot(box, wrc_ref[:, :],
                     preferred_element_type=jnp.float32) + brc_ref[0, :]
    coords_ref[0] = rc[:, :NR]
    classes_ref[0] = rc[:, NR:NR + NC]


@functools.partial(jax.jit, static_argnames=("interpret",))
def _run(flat, xyz, cu, W_out, b_out, W_pos, b_pos, Wq, Wk, Wv, Wo,
         W_cls, b_cls, W_reg, b_reg, interpret=False):
    full = lambda shp: pl.BlockSpec(shp, lambda b: (0,) * len(shp))
    out_specs = (
        pl.BlockSpec((1, L, NR), lambda b: (b, 0, 0)),
        pl.BlockSpec((1, L, NC), lambda b: (b, 0, 0)),
    )
    in_specs = [
        pl.BlockSpec(memory_space=pltpu.SMEM),
        full((N, D)),
        full((N, 3)),
        full((D, D)), full((1, D)),
        full((3, D)), full((1, D)),
        full((D, 3 * D)), full((D, D)),
        full((D, NR + NC)), full((1, NR + NC)),
    ]
    coords, classes = pl.pallas_call(
        _body,
        grid=(B,),
        in_specs=in_specs,
        out_specs=out_specs,
        out_shape=(
            jax.ShapeDtypeStruct((B, L, NR), jnp.float32),
            jax.ShapeDtypeStruct((B, L, NC), jnp.float32),
        ),
        interpret=interpret,
    )(cu, flat, xyz,
      W_out, b_out.reshape(1, D), W_pos, b_pos.reshape(1, D),
      jnp.concatenate([Wq * 0.125, Wk, Wv], axis=1), Wo,
      jnp.concatenate([W_reg, W_cls], axis=1),
      jnp.concatenate([b_reg, b_cls]).reshape(1, NR + NC))
    return coords, classes


def kernel(flat, xyz, keys, cu_seqlens, W_out, b_out, W_pos, b_pos,
           Wq, Wk, Wv, Wo, W_cls, b_cls, W_reg, b_reg):
    del keys
    return _run(flat, xyz, cu_seqlens.astype(jnp.int32), W_out, b_out,
                W_pos, b_pos, Wq, Wk, Wv, Wo, W_cls, b_cls, W_reg, b_reg)


# speedup vs baseline: 1.0321x; 1.0321x over previous
import functools

import jax
import jax.numpy as jnp
from jax.experimental import pallas as pl
from jax.experimental.pallas import tpu as pltpu

B, L, N, D, H, HD = 16, 512, 4096, 256, 4, 64
NC, NR = 18, 6
W = L + 8


def _body(cu_ref, flat_ref, xyz_ref, wout_ref, bout_ref, wpos_ref, bpos_ref,
          wqkv_ref, wo_ref, wrc_ref, brc_ref, coords_ref, classes_ref):
    b = pl.program_id(0)
    s = cu_ref[b]
    e = cu_ref[b + 1]
    ln = e - s

    bpos = bpos_ref[0, :]

    def embed(f, x):
        out = jnp.maximum(
            jax.lax.dot(f, wout_ref[:, :],
                        preferred_element_type=jnp.float32) + bout_ref[0, :],
            0.0)
        pe = jax.lax.dot(x, wpos_ref[:, :], preferred_element_type=jnp.float32)
        return out + pe + bpos

    base = pl.multiple_of(jnp.minimum((s // 8) * 8, N - W), 8)
    d = s - base
    h_win = embed(flat_ref[pl.ds(base, W), :],
                  xyz_ref[pl.ds(base, W), :])
    h_roll = pltpu.roll(h_win, W - d, axis=0)[:L, :]

    row = jax.lax.broadcasted_iota(jnp.int32, (L, 1), 0)
    lcap = jnp.minimum(ln, L)
    valid = row < lcap

    last = jnp.maximum(e - 1, 0)
    l_al = pl.multiple_of((last // 8) * 8, 8)
    sel = jax.lax.broadcasted_iota(jnp.int32, (8, 1), 0) == (last - l_al)
    h8 = embed(flat_ref[pl.ds(l_al, 8), :], xyz_ref[pl.ds(l_al, 8), :])
    h_last = jnp.sum(jnp.where(sel, h8, 0.0), axis=0, keepdims=True)
    repl = jnp.logical_and(ln > L, row == (L - 1))
    h = jnp.where(repl, h_last, jnp.where(valid, h_roll, bpos))

    q = jax.lax.dot(h, wqkv_ref[:, :D], preferred_element_type=jnp.float32)
    k = jax.lax.dot(h, wqkv_ref[:, D:2 * D],
                    preferred_element_type=jnp.float32)
    v = jax.lax.dot(h, wqkv_ref[:, 2 * D:],
                    preferred_element_type=jnp.float32)

    ones_col = jnp.ones((L, 1), jnp.float32)
    kbias = jnp.where(valid, 0.0, -40.0)
    heads = []
    for hh in range(H):
        sl = slice(hh * HD, (hh + 1) * HD)
        qh = jnp.concatenate([q[:, sl], ones_col], axis=1)
        kh = jnp.concatenate([k[:, sl], kbias], axis=1)
        vh = jnp.concatenate([v[:, sl], ones_col], axis=1)
        lg = jax.lax.dot_general(qh, kh, (((1,), (1,)), ((), ())),
                                 preferred_element_type=jnp.float32)
        p = jnp.exp(lg)
        pv = jax.lax.dot(p, vh, preferred_element_type=jnp.float32)
        heads.append(pv[:, :HD] / pv[:, HD:HD + 1])
    ao = jnp.concatenate(heads, axis=-1)
    box = h + jax.lax.dot(ao, wo_ref[:, :], preferred_element_type=jnp.float32)

    coords_ref[0] = jax.lax.dot(
        box, wrc_ref[:, :NR],
        preferred_element_type=jnp.float32) + brc_ref[0, :NR]
    classes_ref[0] = jax.lax.dot(
        box, wrc_ref[:, NR:],
        preferred_element_type=jnp.float32) + brc_ref[0, NR:]


@functools.partial(jax.jit, static_argnames=("interpret",))
def _run(flat, xyz, cu, W_out, b_out, W_pos, b_pos, Wq, Wk, Wv, Wo,
         W_cls, b_cls, W_reg, b_reg, interpret=False):
    full = lambda shp: pl.BlockSpec(shp, lambda b: (0,) * len(shp))
    out_specs = (
        pl.BlockSpec((1, L, NR), lambda b: (b, 0, 0)),
        pl.BlockSpec((1, L, NC), lambda b: (b, 0, 0)),
    )
    in_specs = [
        pl.BlockSpec(memory_space=pltpu.SMEM),
        full((N, D)),
        full((N, 3)),
        full((D, D)), full((1, D)),
        full((3, D)), full((1, D)),
        full((D, 3 * D)), full((D, D)),
        full((D, NR + NC)), full((1, NR + NC)),
    ]
    coords, classes = pl.pallas_call(
        _body,
        grid=(B,),
        in_specs=in_specs,
        out_specs=out_specs,
        out_shape=(
            jax.ShapeDtypeStruct((B, L, NR), jnp.float32),
            jax.ShapeDtypeStruct((B, L, NC), jnp.float32),
        ),
        interpret=interpret,
    )(cu, flat, xyz,
      W_out, b_out.reshape(1, D), W_pos, b_pos.reshape(1, D),
      jnp.concatenate([Wq * 0.125, Wk, Wv], axis=1), Wo,
      jnp.concatenate([W_reg, W_cls], axis=1),
      jnp.concatenate([b_reg, b_cls]).reshape(1, NR + NC))
    return coords, classes


def kernel(flat, xyz, keys, cu_seqlens, W_out, b_out, W_pos, b_pos,
           Wq, Wk, Wv, Wo, W_cls, b_cls, W_reg, b_reg):
    del keys
    return _run(flat, xyz, cu_seqlens.astype(jnp.int32), W_out, b_out,
                W_pos, b_pos, Wq, Wk, Wv, Wo, W_cls, b_cls, W_reg, b_reg)


# speedup vs baseline: 1.0768x; 1.0433x over previous
import functools

import jax
import jax.numpy as jnp
from jax.experimental import pallas as pl
from jax.experimental.pallas import tpu as pltpu

B, L, N, D, H, HD = 16, 512, 4096, 256, 4, 64
NC, NR = 18, 6
W = L + 8


def _body(cu_ref, flat_ref, xyz_ref, wout_ref, bout_ref, wpos_ref, bpos_ref,
          wq_ref, wk_ref, wv_ref, wo_ref, wcls_ref, bcls_ref, wreg_ref,
          breg_ref, coords_ref, classes_ref):
    b = pl.program_id(0)
    s = cu_ref[b]
    e = cu_ref[b + 1]
    ln = e - s

    bpos = bpos_ref[0, :]

    def embed(f, x):
        out = jnp.maximum(
            jax.lax.dot(f, wout_ref[:, :],
                        preferred_element_type=jnp.float32) + bout_ref[0, :],
            0.0)
        pe = jax.lax.dot(x, wpos_ref[:, :], preferred_element_type=jnp.float32)
        return out + pe + bpos

    base = pl.multiple_of(jnp.minimum((s // 8) * 8, N - W), 8)
    d = s - base
    h_win = embed(flat_ref[pl.ds(base, W), :],
                  xyz_ref[pl.ds(base, W), :])
    h_roll = pltpu.roll(h_win, W - d, axis=0)[:L, :]

    row = jax.lax.broadcasted_iota(jnp.int32, (L, 1), 0)
    lcap = jnp.minimum(ln, L)
    valid = row < lcap

    last = jnp.maximum(e - 1, 0)
    l_al = pl.multiple_of((last // 8) * 8, 8)
    sel = jax.lax.broadcasted_iota(jnp.int32, (8, 1), 0) == (last - l_al)
    h8 = embed(flat_ref[pl.ds(l_al, 8), :], xyz_ref[pl.ds(l_al, 8), :])
    h_last = jnp.sum(jnp.where(sel, h8, 0.0), axis=0, keepdims=True)
    repl = jnp.logical_and(ln > L, row == (L - 1))
    h = jnp.where(repl, h_last, jnp.where(valid, h_roll, bpos))

    q = jax.lax.dot(h, wq_ref[:, :],
                    preferred_element_type=jnp.float32) * 0.125
    k = jax.lax.dot(h, wk_ref[:, :], preferred_element_type=jnp.float32)
    v = jax.lax.dot(h, wv_ref[:, :], preferred_element_type=jnp.float32)

    ones_col = jnp.ones((L, 1), jnp.float32)
    kbias = jnp.where(valid, 0.0, -40.0)
    heads = []
    for hh in range(H):
        sl = slice(hh * HD, (hh + 1) * HD)
        qh = jnp.concatenate([q[:, sl], ones_col], axis=1)
        kh = jnp.concatenate([k[:, sl], kbias], axis=1)
        vh = jnp.concatenate([v[:, sl], ones_col], axis=1)
        lg = jax.lax.dot_general(qh, kh, (((1,), (1,)), ((), ())),
                                 preferred_element_type=jnp.float32)
        p = jnp.exp(lg)
        pv = jax.lax.dot(p, vh, preferred_element_type=jnp.float32)
        heads.append(pv[:, :HD] / pv[:, HD:HD + 1])
    ao = jnp.concatenate(heads, axis=-1)
    box = h + jax.lax.dot(ao, wo_ref[:, :], preferred_element_type=jnp.float32)

    coords_ref[0] = jax.lax.dot(
        box, wreg_ref[:, :], preferred_element_type=jnp.float32) + breg_ref[0, :]
    classes_ref[0] = jax.lax.dot(
        box, wcls_ref[:, :], preferred_element_type=jnp.float32) + bcls_ref[0, :]


@functools.partial(jax.jit, static_argnames=("interpret",))
def _run(flat, xyz, cu, W_out, b_out, W_pos, b_pos, Wq, Wk, Wv, Wo,
         W_cls, b_cls, W_reg, b_reg, interpret=False):
    full = lambda shp: pl.BlockSpec(shp, lambda b: (0,) * len(shp))
    out_specs = (
        pl.BlockSpec((1, L, NR), lambda b: (b, 0, 0)),
        pl.BlockSpec((1, L, NC), lambda b: (b, 0, 0)),
    )
    in_specs = [
        pl.BlockSpec(memory_space=pltpu.SMEM),
        full((N, D)),
        full((N, 3)),
        full((D, D)), full((1, D)),
        full((3, D)), full((1, D)),
        full((D, D)), full((D, D)), full((D, D)), full((D, D)),
        full((D, NC)), full((1, NC)),
        full((D, NR)), full((1, NR)),
    ]
    coords, classes = pl.pallas_call(
        _body,
        grid=(B,),
        in_specs=in_specs,
        out_specs=out_specs,
        out_shape=(
            jax.ShapeDtypeStruct((B, L, NR), jnp.float32),
            jax.ShapeDtypeStruct((B, L, NC), jnp.float32),
        ),
        interpret=interpret,
    )(cu, flat, xyz,
      W_out, b_out.reshape(1, D), W_pos, b_pos.reshape(1, D),
      Wq, Wk, Wv, Wo, W_cls, b_cls.reshape(1, NC), W_reg, b_reg.reshape(1, NR))
    return coords, classes


def kernel(flat, xyz, keys, cu_seqlens, W_out, b_out, W_pos, b_pos,
           Wq, Wk, Wv, Wo, W_cls, b_cls, W_reg, b_reg):
    del keys
    return _run(flat, xyz, cu_seqlens.astype(jnp.int32), W_out, b_out,
                W_pos, b_pos, Wq, Wk, Wv, Wo, W_cls, b_cls, W_reg, b_reg)
